# Initial kernel scaffold; baseline (speedup 1.0000x reference)
import jax
import jax.numpy as jnp
from jax.experimental import pallas as pl


def kernel(input_ids, table):
    raise NotImplementedError("write your pallas kernel here")


# speedup vs baseline: 1.2503x; 1.2503x over previous
import functools

import jax
import jax.numpy as jnp
from jax import lax
from jax.experimental import pallas as pl
from jax.experimental.pallas import tpu as pltpu
from jax.experimental.pallas import tpu_sc as plsc

NUM_EMBEDDINGS = 100000
EMBEDDING_DIM = 1024

_NUM_CORES = 2
_NUM_SUBCORES = 16
_NUM_WORKERS = _NUM_CORES * _NUM_SUBCORES

_B_TOTAL = 1024 * 50
_B_PER_W = _B_TOTAL // _NUM_WORKERS
_CHUNK = 40
_N_CHUNKS = _B_PER_W // _CHUNK


def _gather_body(idx_hbm, table_hbm, out_hbm, idx_v, rows_v, sem):
    wid = lax.axis_index("s") * _NUM_CORES + lax.axis_index("c")
    base = wid * _B_PER_W
    pltpu.sync_copy(idx_hbm.at[pl.ds(base, _B_PER_W)], idx_v)

    @pl.loop(0, _N_CHUNKS)
    def _chunk(i):
        start = i * _CHUNK
        pltpu.async_copy(
            table_hbm.at[idx_v.at[pl.ds(start, _CHUNK)]], rows_v, sem
        ).wait()
        pltpu.sync_copy(rows_v, out_hbm.at[pl.ds(base + start, _CHUNK)])


@jax.jit
def _encode(input_ids, table):
    flat_ids = input_ids.reshape(-1)
    mesh = plsc.VectorSubcoreMesh(core_axis_name="c", subcore_axis_name="s")
    out = pl.kernel(
        _gather_body,
        out_type=jax.ShapeDtypeStruct((_B_TOTAL, EMBEDDING_DIM), jnp.float32),
        mesh=mesh,
        scratch_types=[
            pltpu.VMEM((_B_PER_W,), jnp.int32),
            pltpu.VMEM((_CHUNK, EMBEDDING_DIM), jnp.float32),
            pltpu.SemaphoreType.DMA,
        ],
    )(flat_ids, table)
    return out.reshape(input_ids.shape[0], input_ids.shape[1], EMBEDDING_DIM)


def kernel(input_ids, table):
    return _encode(input_ids, table)


# speedup vs baseline: 1.3095x; 1.0473x over previous
import functools

import jax
import jax.numpy as jnp
from jax import lax
from jax.experimental import pallas as pl
from jax.experimental.pallas import tpu as pltpu
from jax.experimental.pallas import tpu_sc as plsc

NUM_EMBEDDINGS = 100000
EMBEDDING_DIM = 1024

_NUM_CORES = 2
_NUM_SUBCORES = 16
_NUM_WORKERS = _NUM_CORES * _NUM_SUBCORES

_B_TOTAL = 1024 * 50
_B_PER_W = _B_TOTAL // _NUM_WORKERS
_CHUNK = 40
_N_CHUNKS = _B_PER_W // _CHUNK


def _gather_body(idx_hbm, table_hbm, out_hbm, idx_v, rows_v, sem0, sem1):
    wid = lax.axis_index("s") * _NUM_CORES + lax.axis_index("c")
    base = wid * _B_PER_W
    pltpu.sync_copy(idx_hbm.at[pl.ds(base, _B_PER_W)], idx_v)

    sems = (sem0, sem1)

    def start_gather(b, c):
        pltpu.async_copy(
            table_hbm.at[idx_v.at[pl.ds(c * _CHUNK, _CHUNK)]],
            rows_v.at[b],
            sems[b],
        )

    def wait_gather(b):
        pltpu.make_async_copy(
            table_hbm.at[pl.ds(0, _CHUNK)], rows_v.at[b], sems[b]
        ).wait()

    start_gather(0, 0)
    start_gather(1, 1)

    @pl.loop(0, _N_CHUNKS - 2, step=2)
    def _chunk(i):
        for b in range(2):
            c = i + b
            wait_gather(b)
            pltpu.sync_copy(rows_v.at[b], out_hbm.at[pl.ds(base + c * _CHUNK, _CHUNK)])
            start_gather(b, c + 2)

    for b in range(2):
        c = _N_CHUNKS - 2 + b
        wait_gather(b)
        pltpu.sync_copy(rows_v.at[b], out_hbm.at[pl.ds(base + c * _CHUNK, _CHUNK)])


@jax.jit
def _encode(input_ids, table):
    flat_ids = input_ids.reshape(-1)
    mesh = plsc.VectorSubcoreMesh(core_axis_name="c", subcore_axis_name="s")
    out = pl.kernel(
        _gather_body,
        out_type=jax.ShapeDtypeStruct((_B_TOTAL, EMBEDDING_DIM), jnp.float32),
        mesh=mesh,
        scratch_types=[
            pltpu.VMEM((_B_PER_W,), jnp.int32),
            pltpu.VMEM((2, _CHUNK, EMBEDDING_DIM), jnp.float32),
            pltpu.SemaphoreType.DMA,
            pltpu.SemaphoreType.DMA,
        ],
    )(flat_ids, table)
    return out.reshape(input_ids.shape[0], input_ids.shape[1], EMBEDDING_DIM)


def kernel(input_ids, table):
    return _encode(input_ids, table)
